# baseline (speedup 1.0000x reference)
import functools

import jax
import jax.numpy as jnp
from jax import lax
from jax.experimental import pallas as pl
from jax.experimental.pallas import tpu as pltpu
from jax.experimental.pallas import tpu_sc as plsc

NUM_ROWS = 1000001
HIDDEN = 64
BATCH = 16384

NC = 2
NS = 16
NW = NC * NS
B_PER_W = BATCH // NW
CHUNK = 128
N_CHUNKS = B_PER_W // CHUNK


def _gather_body(labels_hbm, table_hbm, out_hbm, idx_v, rows_v, sem):
    wid = lax.axis_index("s") * NC + lax.axis_index("c")
    base = wid * B_PER_W
    pltpu.sync_copy(labels_hbm.at[pl.ds(base, B_PER_W)], idx_v)
    copies = []
    for j in range(N_CHUNKS):
        copies.append(
            pltpu.async_copy(
                table_hbm.at[idx_v.at[pl.ds(j * CHUNK, CHUNK)]],
                rows_v.at[pl.ds(j * CHUNK, CHUNK)],
                sem,
            )
        )
    for c in copies:
        c.wait()
    pltpu.sync_copy(rows_v, out_hbm.at[pl.ds(base, B_PER_W)])


@functools.partial(
    pl.kernel,
    out_type=jax.ShapeDtypeStruct((BATCH, HIDDEN), jnp.float32),
    mesh=plsc.VectorSubcoreMesh(core_axis_name="c", subcore_axis_name="s"),
    scratch_types=[
        pltpu.VMEM((B_PER_W,), jnp.int32),
        pltpu.VMEM((B_PER_W, HIDDEN), jnp.float32),
        pltpu.SemaphoreType.DMA,
    ],
    compiler_params=pltpu.CompilerParams(use_tc_tiling_on_sc=False),
)
def _embed_lookup(labels_hbm, table_hbm, out_hbm, idx_v, rows_v, sem):
    _gather_body(labels_hbm, table_hbm, out_hbm, idx_v, rows_v, sem)


def kernel(labels, train, table):
    embeddings = _embed_lookup(labels.astype(jnp.int32), table)
    return (embeddings, labels)


# speedup vs baseline: 1.7206x; 1.7206x over previous
import functools

import jax
import jax.numpy as jnp
from jax import lax
from jax.experimental import pallas as pl
from jax.experimental.pallas import tpu as pltpu
from jax.experimental.pallas import tpu_sc as plsc

NUM_ROWS = 1000001
HIDDEN = 64
BATCH = 16384

NC = 2
NS = 16
NW = NC * NS
B_PER_W = BATCH // NW


def _gather_body(labels_hbm, table_hbm, out_hbm, idx_v, rows_v, sem):
    wid = lax.axis_index("s") * NC + lax.axis_index("c")
    base = wid * B_PER_W
    pltpu.sync_copy(labels_hbm.at[pl.ds(base, B_PER_W)], idx_v)

    def fetch_group(g, carry):
        vec = idx_v[pl.ds(g * 16, 16)]
        for k in range(16):
            i = vec[k]
            pltpu.make_async_copy(
                table_hbm.at[pl.ds(i, 1)],
                rows_v.at[pl.ds(g * 16 + k, 1)],
                sem,
            ).start()
        return carry

    lax.fori_loop(0, B_PER_W // 16, fetch_group, 0)

    pltpu.make_async_copy(
        table_hbm.at[pl.ds(0, B_PER_W)], rows_v, sem
    ).wait()

    pltpu.sync_copy(rows_v, out_hbm.at[pl.ds(base, B_PER_W)])


@functools.partial(
    pl.kernel,
    out_type=jax.ShapeDtypeStruct((BATCH, HIDDEN), jnp.float32),
    mesh=plsc.VectorSubcoreMesh(core_axis_name="c", subcore_axis_name="s"),
    scratch_types=[
        pltpu.VMEM((B_PER_W,), jnp.int32),
        pltpu.VMEM((B_PER_W, HIDDEN), jnp.float32),
        pltpu.SemaphoreType.DMA,
    ],
)
def _embed_lookup(labels_hbm, table_hbm, out_hbm, idx_v, rows_v, sem):
    _gather_body(labels_hbm, table_hbm, out_hbm, idx_v, rows_v, sem)


def kernel(labels, train, table):
    embeddings = _embed_lookup(labels.astype(jnp.int32), table)
    return (embeddings, labels)
